# baseline (speedup 1.0000x reference)
import jax
import jax.numpy as jnp
import numpy as np
from jax import lax
from jax.experimental import pallas as pl
from jax.experimental.pallas import tpu as pltpu
from jax.experimental.pallas import tpu_sc as plsc

NC, NS, L = 2, 16, 16
NW = NC * NS
CHUNK = 128
CH0_FRAC = 93.0 / 157.0


def _chunk_off(cid, sid, ch0, ch1):
    return sid * (ch0 + ch1) + cid * ch0


def _spmm_mesh():
    return plsc.VectorSubcoreMesh(
        core_axis_name="c", subcore_axis_name="s",
        num_cores=NC, num_subcores=NS)


def _zero2d(ref, nrows):
    zero = jnp.zeros((L,), jnp.float32)

    def zb(i, carry):
        r = i // (128 // L)
        g = i - r * (128 // L)
        ref[r, pl.ds(g * L, L)] = zero
        return carry
    lax.fori_loop(0, nrows * (128 // L), zb, 0)


def _deg_body(n_pad, ch0, ch1, col_hbm, deg_out, cidx_v, deg_v):
    cid = lax.axis_index("c")
    sid = lax.axis_index("s")
    wid = sid * NC + cid
    chw = jnp.where(cid == 0, ch0, ch1)
    roff = _chunk_off(cid, sid, ch0, ch1)
    pltpu.sync_copy(col_hbm.at[pl.ds(roff, cidx_v.shape[0])], cidx_v)
    _zero2d(deg_v, n_pad // 128)
    ones = jnp.ones((L,), jnp.float32)
    seven = jnp.full((L,), 7, jnp.int32)
    low = jnp.full((L,), 127, jnp.int32)

    def chunk(j, carry):
        for g in range(CHUNK // L):
            cv = cidx_v[j, pl.ds(g * L, L)]
            plsc.addupdate_scatter(deg_v, [cv >> seven, cv & low], ones)
        return carry
    lax.fori_loop(0, chw, chunk, 0)
    pltpu.sync_copy(deg_v, deg_out.at[wid])


def _spmm1_body(n_pad, ch0, ch1,
                xws_lo_hbm, xws_hi_hbm, row_hbm, col_hbm, dinv_hbm, zeros_hbm,
                acc_lo_out, acc_hi_out, t_out,
                ridx_v, cidx_v, dinv_v, t_v, rows_v, acc_s, sem):
    cid = lax.axis_index("c")
    sid = lax.axis_index("s")
    wid = sid * NC + cid
    chw = jnp.where(cid == 0, ch0, ch1)
    roff = _chunk_off(cid, sid, ch0, ch1)
    rpt = n_pad // NS
    my = pl.ds(sid * rpt, rpt)
    pltpu.sync_copy(row_hbm.at[pl.ds(roff, ridx_v.shape[0])], ridx_v)
    pltpu.sync_copy(col_hbm.at[pl.ds(roff, cidx_v.shape[0])], cidx_v)
    pltpu.sync_copy(dinv_hbm, dinv_v)
    _zero2d(t_v, n_pad // 128)
    seven = jnp.full((L,), 7, jnp.int32)
    low = jnp.full((L,), 127, jnp.int32)

    for p, (xws_hbm, acc_out) in enumerate(
            ((xws_lo_hbm, acc_lo_out), (xws_hi_hbm, acc_hi_out))):
        pltpu.sync_copy(zeros_hbm, acc_s.at[my])
        plsc.subcore_barrier()

        def chunk(j, carry):
            pltpu.async_copy(xws_hbm.at[ridx_v.at[j]], rows_v, sem).wait()
            pltpu.sync_copy(rows_v, acc_s.at[cidx_v.at[j]], add=True)
            if p == 0:
                for g in range(CHUNK // L):
                    rv = ridx_v[j, pl.ds(g * L, L)]
                    cv = cidx_v[j, pl.ds(g * L, L)]
                    dvals = plsc.load_gather(dinv_v, [rv >> seven, rv & low])
                    plsc.addupdate_scatter(t_v, [cv >> seven, cv & low],
                                           dvals)
            return carry
        lax.fori_loop(0, chw, chunk, 0)
        plsc.subcore_barrier()
        pltpu.sync_copy(acc_s.at[my], acc_out.at[cid, my])
    pltpu.sync_copy(t_v, t_out.at[wid])


def _spmm2_body(n_pad, ch0, ch1,
                us_hbm, row_hbm, col_hbm, o_out,
                us_v, ridx_v, cidx_v, o0_v, o1_v):
    cid = lax.axis_index("c")
    sid = lax.axis_index("s")
    wid = sid * NC + cid
    chw = jnp.where(cid == 0, ch0, ch1)
    roff = _chunk_off(cid, sid, ch0, ch1)
    pltpu.sync_copy(us_hbm, us_v)
    pltpu.sync_copy(row_hbm.at[pl.ds(roff, ridx_v.shape[0])], ridx_v)
    pltpu.sync_copy(col_hbm.at[pl.ds(roff, cidx_v.shape[0])], cidx_v)
    _zero2d(o0_v, n_pad // 128)
    _zero2d(o1_v, n_pad // 128)

    two = jnp.full((L,), 2, jnp.int32)
    one = jnp.full((L,), 1, jnp.int32)
    seven = jnp.full((L,), 7, jnp.int32)
    low = jnp.full((L,), 127, jnp.int32)

    def chunk(j, carry):
        for g in range(CHUNK // L):
            rv = ridx_v[j, pl.ds(g * L, L)]
            cv = cidx_v[j, pl.ds(g * L, L)]
            i0 = rv * two
            v0 = plsc.load_gather(us_v, [i0])
            v1 = plsc.load_gather(us_v, [i0 + one])
            chi = cv >> seven
            clo = cv & low
            plsc.addupdate_scatter(o0_v, [chi, clo], v0)
            plsc.addupdate_scatter(o1_v, [chi, clo], v1)
        return carry
    lax.fori_loop(0, chw, chunk, 0)
    pltpu.sync_copy(o0_v, o_out.at[wid, 0])
    pltpu.sync_copy(o1_v, o_out.at[wid, 1])


def _dinv_body(degp_ref, dinv_ref):
    deg = jnp.sum(degp_ref[...], axis=0)
    dinv_ref[...] = jnp.where(deg > 0, lax.rsqrt(deg), 0.0)


def _xws_body(n_real, x_ref, dinvc_ref, w1a_ref, w1b_ref,
              xws_lo_ref, xws_hi_ref, c_ref, sum_ref):
    i = pl.program_id(0)
    xb = x_ref[...]
    xw = jnp.dot(xb, w1a_ref[...], preferred_element_type=jnp.float32)
    xws = dinvc_ref[...] * xw
    half = xws.shape[1] // 2
    xws_lo_ref[...] = xws[:, :half]
    xws_hi_ref[...] = xws[:, half:]
    s = jnp.sum(xb, axis=0, keepdims=True)

    @pl.when(i == 0)
    def _():
        sum_ref[...] = s

    @pl.when(i > 0)
    def _():
        sum_ref[...] = sum_ref[...] + s

    @pl.when(i == pl.num_programs(0) - 1)
    def _():
        c_ref[...] = jnp.dot(sum_ref[...] * (1.0 / n_real), w1b_ref[...],
                             preferred_element_type=jnp.float32)


def _dt_body(tp_ref, dinv_ref, dt_ref):
    dt_ref[...] = dinv_ref[...] * jnp.sum(tp_ref[...], axis=0)


def _h1_body(alo0_ref, alo1_ref, ahi0_ref, ahi1_ref, dinvc_ref, dtc_ref,
             c_ref, b1_ref, w2a_ref, w2b_ref, us_ref):
    d = dinvc_ref[...]
    dtc = dtc_ref[...]
    cv = c_ref[...]
    b1v = b1_ref[...]
    half = cv.shape[1] // 2
    h1_lo = jnp.maximum(
        d * (alo0_ref[...] + alo1_ref[...]) + dtc * cv[:, :half]
        + b1v[:, :half], 0.0)
    h1_hi = jnp.maximum(
        d * (ahi0_ref[...] + ahi1_ref[...]) + dtc * cv[:, half:]
        + b1v[:, half:], 0.0)
    u = (jnp.dot(h1_lo, w2a_ref[...], preferred_element_type=jnp.float32)
         + jnp.dot(h1_hi, w2b_ref[...], preferred_element_type=jnp.float32))
    us_ref[...] = d * u


def _final_body(op_ref, dinv_ref, b2b_ref, out_ref):
    s = jnp.sum(op_ref[...], axis=0)
    d = dinv_ref[...]
    out_ref[...] = d[None] * s + b2b_ref[...]


def kernel(x, edge_index, W1, b1, W2, b2):
    n, f = x.shape
    hid = W1.shape[1]
    e = edge_index.shape[1]

    n_pad = ((n + 1 + 255) // 256) * 256
    rb = 2048
    nrows = n_pad // 128

    ch_sum = -(-e // (NS * CHUNK))
    ch0 = int(round(ch_sum * CH0_FRAC))
    ch1 = ch_sum - ch0
    ch_max = max(ch0, ch1)
    tot_rows = NS * (ch0 + ch1) + ch_max
    npad_e = tot_rows * CHUNK - e

    def _layout(v):
        return jnp.concatenate(
            [v, jnp.full((npad_e,), n, jnp.int32)]).reshape(tot_rows, CHUNK)

    row_w = _layout(edge_index[0].astype(jnp.int32))
    col_w = _layout(edge_index[1].astype(jnp.int32))
    x_pad = jnp.concatenate([x, jnp.zeros((n_pad - n, f), x.dtype)])
    W1a, W1b = W1[:f], W1[f:]
    half = hid // 2
    W2a, W2b = W2[:half], W2[half:]
    zeros_blk = jnp.zeros((n_pad // NS, half), jnp.float32)

    mesh = _spmm_mesh()
    sc_params = pltpu.CompilerParams(needs_layout_passes=False,
                                     use_tc_tiling_on_sc=False)

    deg_part = pl.kernel(
        lambda *a: _deg_body(n_pad, ch0, ch1, *a),
        out_type=jax.ShapeDtypeStruct((NW, nrows, 128), jnp.float32),
        mesh=mesh,
        scratch_types=[pltpu.VMEM((ch_max, CHUNK), jnp.int32),
                       pltpu.VMEM((nrows, 128), jnp.float32)],
        compiler_params=sc_params,
    )(col_w)

    dinv = pl.pallas_call(
        _dinv_body,
        out_shape=jax.ShapeDtypeStruct((nrows, 128), jnp.float32),
    )(deg_part)
    dinv_col = dinv.reshape(n_pad, 1)

    xws_lo, xws_hi, c = pl.pallas_call(
        lambda *a: _xws_body(float(n), *a),
        grid=(n_pad // rb,),
        in_specs=[pl.BlockSpec((rb, f), lambda i: (i, 0)),
                  pl.BlockSpec((rb, 1), lambda i: (i, 0)),
                  pl.BlockSpec((f, hid), lambda i: (0, 0)),
                  pl.BlockSpec((f, hid), lambda i: (0, 0))],
        out_specs=[pl.BlockSpec((rb, half), lambda i: (i, 0)),
                   pl.BlockSpec((rb, half), lambda i: (i, 0)),
                   pl.BlockSpec((1, hid), lambda i: (0, 0))],
        out_shape=[jax.ShapeDtypeStruct((n_pad, half), jnp.float32),
                   jax.ShapeDtypeStruct((n_pad, half), jnp.float32),
                   jax.ShapeDtypeStruct((1, hid), jnp.float32)],
        scratch_shapes=[pltpu.VMEM((1, f), jnp.float32)],
    )(x_pad, dinv_col, W1a, W1b)

    acc_lo, acc_hi, t_part = pl.kernel(
        lambda *a: _spmm1_body(n_pad, ch0, ch1, *a),
        out_type=(jax.ShapeDtypeStruct((NC, n_pad, half), jnp.float32),
                  jax.ShapeDtypeStruct((NC, n_pad, half), jnp.float32),
                  jax.ShapeDtypeStruct((NW, nrows, 128), jnp.float32)),
        mesh=mesh,
        scratch_types=[pltpu.VMEM((ch_max, CHUNK), jnp.int32),
                       pltpu.VMEM((ch_max, CHUNK), jnp.int32),
                       pltpu.VMEM((nrows, 128), jnp.float32),
                       pltpu.VMEM((nrows, 128), jnp.float32),
                       pltpu.VMEM((CHUNK, half), jnp.float32),
                       pltpu.VMEM_SHARED((n_pad, half), jnp.float32),
                       pltpu.SemaphoreType.DMA],
        compiler_params=sc_params,
    )(xws_lo, xws_hi, row_w, col_w, dinv, zeros_blk)

    dt = pl.pallas_call(
        _dt_body,
        out_shape=jax.ShapeDtypeStruct((nrows, 128), jnp.float32),
    )(t_part, dinv)

    us = pl.pallas_call(
        _h1_body,
        grid=(n_pad // rb,),
        in_specs=[pl.BlockSpec((rb, half), lambda i: (i, 0)),
                  pl.BlockSpec((rb, half), lambda i: (i, 0)),
                  pl.BlockSpec((rb, half), lambda i: (i, 0)),
                  pl.BlockSpec((rb, half), lambda i: (i, 0)),
                  pl.BlockSpec((rb, 1), lambda i: (i, 0)),
                  pl.BlockSpec((rb, 1), lambda i: (i, 0)),
                  pl.BlockSpec((1, hid), lambda i: (0, 0)),
                  pl.BlockSpec((1, hid), lambda i: (0, 0)),
                  pl.BlockSpec((half, 2), lambda i: (0, 0)),
                  pl.BlockSpec((half, 2), lambda i: (0, 0))],
        out_specs=pl.BlockSpec((rb, 2), lambda i: (i, 0)),
        out_shape=jax.ShapeDtypeStruct((n_pad, 2), jnp.float32),
    )(acc_lo[0], acc_lo[1], acc_hi[0], acc_hi[1], dinv_col,
      dt.reshape(n_pad, 1), c, b1.reshape(1, hid), W2a, W2b)

    o_part = pl.kernel(
        lambda *a: _spmm2_body(n_pad, ch0, ch1, *a),
        out_type=jax.ShapeDtypeStruct((NW, 2, nrows, 128), jnp.float32),
        mesh=mesh,
        scratch_types=[pltpu.VMEM((2 * n_pad,), jnp.float32),
                       pltpu.VMEM((ch_max, CHUNK), jnp.int32),
                       pltpu.VMEM((ch_max, CHUNK), jnp.int32),
                       pltpu.VMEM((nrows, 128), jnp.float32),
                       pltpu.VMEM((nrows, 128), jnp.float32)],
        compiler_params=sc_params,
    )(us.reshape(2 * n_pad), row_w, col_w)

    b2b = jnp.broadcast_to(b2.reshape(2, 1, 1), (2, nrows, 128))
    out2 = pl.pallas_call(
        _final_body,
        out_shape=jax.ShapeDtypeStruct((2, nrows, 128), jnp.float32),
    )(o_part, dinv, b2b)

    return out2.reshape(2, n_pad)[:, :n].T


# speedup vs baseline: 1.0215x; 1.0215x over previous
import jax
import jax.numpy as jnp
import numpy as np
from jax import lax
from jax.experimental import pallas as pl
from jax.experimental.pallas import tpu as pltpu
from jax.experimental.pallas import tpu_sc as plsc

NC, NS, L = 2, 16, 16
NW = NC * NS
CHUNK = 128
CH0_FRAC = 90.0 / 157.0


def _chunk_off(cid, sid, ch0, ch1):
    return sid * (ch0 + ch1) + cid * ch0


def _spmm_mesh():
    return plsc.VectorSubcoreMesh(
        core_axis_name="c", subcore_axis_name="s",
        num_cores=NC, num_subcores=NS)


def _zero2d(ref, nrows):
    zero = jnp.zeros((L,), jnp.float32)

    def zb(i, carry):
        r = i // (128 // L)
        g = i - r * (128 // L)
        ref[r, pl.ds(g * L, L)] = zero
        return carry
    lax.fori_loop(0, nrows * (128 // L), zb, 0)


def _deg_body(n_pad, ch0, ch1, col_hbm, deg_out, cidx_v, deg_v):
    cid = lax.axis_index("c")
    sid = lax.axis_index("s")
    wid = sid * NC + cid
    chw = jnp.where(cid == 0, ch0, ch1)
    roff = _chunk_off(cid, sid, ch0, ch1)
    pltpu.sync_copy(col_hbm.at[pl.ds(roff, cidx_v.shape[0])], cidx_v)
    _zero2d(deg_v, n_pad // 128)
    ones = jnp.ones((L,), jnp.float32)
    seven = jnp.full((L,), 7, jnp.int32)
    low = jnp.full((L,), 127, jnp.int32)

    def chunk(j, carry):
        for g in range(CHUNK // L):
            cv = cidx_v[j, pl.ds(g * L, L)]
            plsc.addupdate_scatter(deg_v, [cv >> seven, cv & low], ones)
        return carry
    lax.fori_loop(0, chw, chunk, 0)
    pltpu.sync_copy(deg_v, deg_out.at[wid])


def _spmm1_body(n_pad, ch0, ch1,
                xws_lo_hbm, xws_hi_hbm, row_hbm, col_hbm, dinv_hbm, zeros_hbm,
                acc_lo_out, acc_hi_out, t_out,
                ridx_v, cidx_v, dinv_v, t_v, rows_v, acc_s, sem):
    cid = lax.axis_index("c")
    sid = lax.axis_index("s")
    wid = sid * NC + cid
    chw = jnp.where(cid == 0, ch0, ch1)
    roff = _chunk_off(cid, sid, ch0, ch1)
    rpt = n_pad // NS
    my = pl.ds(sid * rpt, rpt)
    pltpu.sync_copy(row_hbm.at[pl.ds(roff, ridx_v.shape[0])], ridx_v)
    pltpu.sync_copy(col_hbm.at[pl.ds(roff, cidx_v.shape[0])], cidx_v)
    pltpu.sync_copy(dinv_hbm, dinv_v)
    _zero2d(t_v, n_pad // 128)
    seven = jnp.full((L,), 7, jnp.int32)
    low = jnp.full((L,), 127, jnp.int32)

    for p, (xws_hbm, acc_out) in enumerate(
            ((xws_lo_hbm, acc_lo_out), (xws_hi_hbm, acc_hi_out))):
        pltpu.sync_copy(zeros_hbm, acc_s.at[my])
        plsc.subcore_barrier()

        def chunk(j, carry):
            pltpu.async_copy(xws_hbm.at[ridx_v.at[j]], rows_v, sem).wait()
            pltpu.sync_copy(rows_v, acc_s.at[cidx_v.at[j]], add=True)
            if p == 0:
                for g in range(CHUNK // L):
                    rv = ridx_v[j, pl.ds(g * L, L)]
                    cv = cidx_v[j, pl.ds(g * L, L)]
                    dvals = plsc.load_gather(dinv_v, [rv >> seven, rv & low])
                    plsc.addupdate_scatter(t_v, [cv >> seven, cv & low],
                                           dvals)
            return carry
        lax.fori_loop(0, chw, chunk, 0)
        plsc.subcore_barrier()
        pltpu.sync_copy(acc_s.at[my], acc_out.at[cid, my])
    pltpu.sync_copy(t_v, t_out.at[wid])


def _spmm2_body(n_pad, ch0, ch1,
                us_hbm, row_hbm, col_hbm, o_out,
                us_v, ridx_v, cidx_v, o0_v, o1_v):
    cid = lax.axis_index("c")
    sid = lax.axis_index("s")
    wid = sid * NC + cid
    chw = jnp.where(cid == 0, ch0, ch1)
    roff = _chunk_off(cid, sid, ch0, ch1)
    pltpu.sync_copy(us_hbm, us_v)
    pltpu.sync_copy(row_hbm.at[pl.ds(roff, ridx_v.shape[0])], ridx_v)
    pltpu.sync_copy(col_hbm.at[pl.ds(roff, cidx_v.shape[0])], cidx_v)
    _zero2d(o0_v, n_pad // 128)
    _zero2d(o1_v, n_pad // 128)

    two = jnp.full((L,), 2, jnp.int32)
    one = jnp.full((L,), 1, jnp.int32)
    seven = jnp.full((L,), 7, jnp.int32)
    low = jnp.full((L,), 127, jnp.int32)

    def chunk(j, carry):
        for g in range(CHUNK // L):
            rv = ridx_v[j, pl.ds(g * L, L)]
            cv = cidx_v[j, pl.ds(g * L, L)]
            i0 = rv * two
            v0 = plsc.load_gather(us_v, [i0])
            v1 = plsc.load_gather(us_v, [i0 + one])
            chi = cv >> seven
            clo = cv & low
            plsc.addupdate_scatter(o0_v, [chi, clo], v0)
            plsc.addupdate_scatter(o1_v, [chi, clo], v1)
        return carry
    lax.fori_loop(0, chw, chunk, 0)
    pltpu.sync_copy(o0_v, o_out.at[wid, 0])
    pltpu.sync_copy(o1_v, o_out.at[wid, 1])


def _dinv_body(degp_ref, dinv_ref):
    deg = jnp.sum(degp_ref[...], axis=0)
    dinv_ref[...] = jnp.where(deg > 0, lax.rsqrt(deg), 0.0)


def _xws_body(n_real, x_ref, dinvc_ref, w1a_ref, w1b_ref,
              xws_lo_ref, xws_hi_ref, c_ref, sum_ref):
    i = pl.program_id(0)
    xb = x_ref[...]
    xw = jnp.dot(xb, w1a_ref[...], preferred_element_type=jnp.float32)
    xws = dinvc_ref[...] * xw
    half = xws.shape[1] // 2
    xws_lo_ref[...] = xws[:, :half]
    xws_hi_ref[...] = xws[:, half:]
    s = jnp.sum(xb, axis=0, keepdims=True)

    @pl.when(i == 0)
    def _():
        sum_ref[...] = s

    @pl.when(i > 0)
    def _():
        sum_ref[...] = sum_ref[...] + s

    @pl.when(i == pl.num_programs(0) - 1)
    def _():
        c_ref[...] = jnp.dot(sum_ref[...] * (1.0 / n_real), w1b_ref[...],
                             preferred_element_type=jnp.float32)


def _dt_body(tp_ref, dinv_ref, dt_ref):
    dt_ref[...] = dinv_ref[...] * jnp.sum(tp_ref[...], axis=0)


def _h1_body(alo0_ref, alo1_ref, ahi0_ref, ahi1_ref, dinvc_ref, dtc_ref,
             c_ref, b1_ref, w2a_ref, w2b_ref, us_ref):
    d = dinvc_ref[...]
    dtc = dtc_ref[...]
    cv = c_ref[...]
    b1v = b1_ref[...]
    half = cv.shape[1] // 2
    h1_lo = jnp.maximum(
        d * (alo0_ref[...] + alo1_ref[...]) + dtc * cv[:, :half]
        + b1v[:, :half], 0.0)
    h1_hi = jnp.maximum(
        d * (ahi0_ref[...] + ahi1_ref[...]) + dtc * cv[:, half:]
        + b1v[:, half:], 0.0)
    u = (jnp.dot(h1_lo, w2a_ref[...], preferred_element_type=jnp.float32)
         + jnp.dot(h1_hi, w2b_ref[...], preferred_element_type=jnp.float32))
    us_ref[...] = d * u


def _final_body(op_ref, dinv_ref, b2b_ref, out_ref):
    s = jnp.sum(op_ref[...], axis=0)
    d = dinv_ref[...]
    out_ref[...] = d[None] * s + b2b_ref[...]


def kernel(x, edge_index, W1, b1, W2, b2):
    n, f = x.shape
    hid = W1.shape[1]
    e = edge_index.shape[1]

    n_pad = ((n + 1 + 255) // 256) * 256
    rb = 2048
    nrows = n_pad // 128

    ch_sum = -(-e // (NS * CHUNK))
    ch0 = int(round(ch_sum * CH0_FRAC))
    ch1 = ch_sum - ch0
    ch_max = max(ch0, ch1)
    tot_rows = NS * (ch0 + ch1) + ch_max
    npad_e = tot_rows * CHUNK - e

    def _layout(v):
        return jnp.concatenate(
            [v, jnp.full((npad_e,), n, jnp.int32)]).reshape(tot_rows, CHUNK)

    row_w = _layout(edge_index[0].astype(jnp.int32))
    col_w = _layout(edge_index[1].astype(jnp.int32))
    x_pad = jnp.concatenate([x, jnp.zeros((n_pad - n, f), x.dtype)])
    W1a, W1b = W1[:f], W1[f:]
    half = hid // 2
    W2a, W2b = W2[:half], W2[half:]
    zeros_blk = jnp.zeros((n_pad // NS, half), jnp.float32)

    mesh = _spmm_mesh()
    sc_params = pltpu.CompilerParams(needs_layout_passes=False,
                                     use_tc_tiling_on_sc=False)

    deg_part = pl.kernel(
        lambda *a: _deg_body(n_pad, ch0, ch1, *a),
        out_type=jax.ShapeDtypeStruct((NW, nrows, 128), jnp.float32),
        mesh=mesh,
        scratch_types=[pltpu.VMEM((ch_max, CHUNK), jnp.int32),
                       pltpu.VMEM((nrows, 128), jnp.float32)],
        compiler_params=sc_params,
    )(col_w)

    dinv = pl.pallas_call(
        _dinv_body,
        out_shape=jax.ShapeDtypeStruct((nrows, 128), jnp.float32),
    )(deg_part)
    dinv_col = dinv.reshape(n_pad, 1)

    xws_lo, xws_hi, c = pl.pallas_call(
        lambda *a: _xws_body(float(n), *a),
        grid=(n_pad // rb,),
        in_specs=[pl.BlockSpec((rb, f), lambda i: (i, 0)),
                  pl.BlockSpec((rb, 1), lambda i: (i, 0)),
                  pl.BlockSpec((f, hid), lambda i: (0, 0)),
                  pl.BlockSpec((f, hid), lambda i: (0, 0))],
        out_specs=[pl.BlockSpec((rb, half), lambda i: (i, 0)),
                   pl.BlockSpec((rb, half), lambda i: (i, 0)),
                   pl.BlockSpec((1, hid), lambda i: (0, 0))],
        out_shape=[jax.ShapeDtypeStruct((n_pad, half), jnp.float32),
                   jax.ShapeDtypeStruct((n_pad, half), jnp.float32),
                   jax.ShapeDtypeStruct((1, hid), jnp.float32)],
        scratch_shapes=[pltpu.VMEM((1, f), jnp.float32)],
    )(x_pad, dinv_col, W1a, W1b)

    acc_lo, acc_hi, t_part = pl.kernel(
        lambda *a: _spmm1_body(n_pad, ch0, ch1, *a),
        out_type=(jax.ShapeDtypeStruct((NC, n_pad, half), jnp.float32),
                  jax.ShapeDtypeStruct((NC, n_pad, half), jnp.float32),
                  jax.ShapeDtypeStruct((NW, nrows, 128), jnp.float32)),
        mesh=mesh,
        scratch_types=[pltpu.VMEM((ch_max, CHUNK), jnp.int32),
                       pltpu.VMEM((ch_max, CHUNK), jnp.int32),
                       pltpu.VMEM((nrows, 128), jnp.float32),
                       pltpu.VMEM((nrows, 128), jnp.float32),
                       pltpu.VMEM((CHUNK, half), jnp.float32),
                       pltpu.VMEM_SHARED((n_pad, half), jnp.float32),
                       pltpu.SemaphoreType.DMA],
        compiler_params=sc_params,
    )(xws_lo, xws_hi, row_w, col_w, dinv, zeros_blk)

    dt = pl.pallas_call(
        _dt_body,
        out_shape=jax.ShapeDtypeStruct((nrows, 128), jnp.float32),
    )(t_part, dinv)

    us = pl.pallas_call(
        _h1_body,
        grid=(n_pad // rb,),
        in_specs=[pl.BlockSpec((rb, half), lambda i: (i, 0)),
                  pl.BlockSpec((rb, half), lambda i: (i, 0)),
                  pl.BlockSpec((rb, half), lambda i: (i, 0)),
                  pl.BlockSpec((rb, half), lambda i: (i, 0)),
                  pl.BlockSpec((rb, 1), lambda i: (i, 0)),
                  pl.BlockSpec((rb, 1), lambda i: (i, 0)),
                  pl.BlockSpec((1, hid), lambda i: (0, 0)),
                  pl.BlockSpec((1, hid), lambda i: (0, 0)),
                  pl.BlockSpec((half, 2), lambda i: (0, 0)),
                  pl.BlockSpec((half, 2), lambda i: (0, 0))],
        out_specs=pl.BlockSpec((rb, 2), lambda i: (i, 0)),
        out_shape=jax.ShapeDtypeStruct((n_pad, 2), jnp.float32),
    )(acc_lo[0], acc_lo[1], acc_hi[0], acc_hi[1], dinv_col,
      dt.reshape(n_pad, 1), c, b1.reshape(1, hid), W2a, W2b)

    o_part = pl.kernel(
        lambda *a: _spmm2_body(n_pad, ch0, ch1, *a),
        out_type=jax.ShapeDtypeStruct((NW, 2, nrows, 128), jnp.float32),
        mesh=mesh,
        scratch_types=[pltpu.VMEM((2 * n_pad,), jnp.float32),
                       pltpu.VMEM((ch_max, CHUNK), jnp.int32),
                       pltpu.VMEM((ch_max, CHUNK), jnp.int32),
                       pltpu.VMEM((nrows, 128), jnp.float32),
                       pltpu.VMEM((nrows, 128), jnp.float32)],
        compiler_params=sc_params,
    )(us.reshape(2 * n_pad), row_w, col_w)

    b2b = jnp.broadcast_to(b2.reshape(2, 1, 1), (2, nrows, 128))
    out2 = pl.pallas_call(
        _final_body,
        out_shape=jax.ShapeDtypeStruct((2, nrows, 128), jnp.float32),
    )(o_part, dinv, b2b)

    return out2.reshape(2, n_pad)[:, :n].T


# speedup vs baseline: 1.0232x; 1.0017x over previous
import jax
import jax.numpy as jnp
import numpy as np
from jax import lax
from jax.experimental import pallas as pl
from jax.experimental.pallas import tpu as pltpu
from jax.experimental.pallas import tpu_sc as plsc

NC, NS, L = 2, 16, 16
NW = NC * NS
CHUNK = 128
CH0_FRAC = 88.0 / 157.0


def _chunk_off(cid, sid, ch0, ch1):
    return sid * (ch0 + ch1) + cid * ch0


def _spmm_mesh():
    return plsc.VectorSubcoreMesh(
        core_axis_name="c", subcore_axis_name="s",
        num_cores=NC, num_subcores=NS)


def _zero2d(ref, nrows):
    zero = jnp.zeros((L,), jnp.float32)

    def zb(i, carry):
        r = i // (128 // L)
        g = i - r * (128 // L)
        ref[r, pl.ds(g * L, L)] = zero
        return carry
    lax.fori_loop(0, nrows * (128 // L), zb, 0)


def _deg_body(n_pad, ch0, ch1, col_hbm, deg_out, cidx_v, deg_v):
    cid = lax.axis_index("c")
    sid = lax.axis_index("s")
    wid = sid * NC + cid
    chw = jnp.where(cid == 0, ch0, ch1)
    roff = _chunk_off(cid, sid, ch0, ch1)
    pltpu.sync_copy(col_hbm.at[pl.ds(roff, cidx_v.shape[0])], cidx_v)
    _zero2d(deg_v, n_pad // 128)
    ones = jnp.ones((L,), jnp.float32)
    seven = jnp.full((L,), 7, jnp.int32)
    low = jnp.full((L,), 127, jnp.int32)

    def chunk(j, carry):
        for g in range(CHUNK // L):
            cv = cidx_v[j, pl.ds(g * L, L)]
            plsc.addupdate_scatter(deg_v, [cv >> seven, cv & low], ones)
        return carry
    lax.fori_loop(0, chw, chunk, 0)
    pltpu.sync_copy(deg_v, deg_out.at[wid])


def _spmm1_body(n_pad, ch0, ch1,
                xws_lo_hbm, xws_hi_hbm, row_hbm, col_hbm, dinv_hbm, zeros_hbm,
                acc_lo_out, acc_hi_out, t_out,
                ridx_v, cidx_v, dinv_v, t_v, rows_v, acc_s, sem):
    cid = lax.axis_index("c")
    sid = lax.axis_index("s")
    wid = sid * NC + cid
    chw = jnp.where(cid == 0, ch0, ch1)
    roff = _chunk_off(cid, sid, ch0, ch1)
    rpt = n_pad // NS
    my = pl.ds(sid * rpt, rpt)
    pltpu.sync_copy(row_hbm.at[pl.ds(roff, ridx_v.shape[0])], ridx_v)
    pltpu.sync_copy(col_hbm.at[pl.ds(roff, cidx_v.shape[0])], cidx_v)
    pltpu.sync_copy(dinv_hbm, dinv_v)
    _zero2d(t_v, n_pad // 128)
    seven = jnp.full((L,), 7, jnp.int32)
    low = jnp.full((L,), 127, jnp.int32)

    for p, (xws_hbm, acc_out) in enumerate(
            ((xws_lo_hbm, acc_lo_out), (xws_hi_hbm, acc_hi_out))):
        pltpu.sync_copy(zeros_hbm, acc_s.at[my])
        plsc.subcore_barrier()

        def chunk(j, carry):
            pltpu.async_copy(xws_hbm.at[ridx_v.at[j]], rows_v, sem).wait()
            pltpu.sync_copy(rows_v, acc_s.at[cidx_v.at[j]], add=True)
            if p == 0:
                for g in range(CHUNK // L):
                    rv = ridx_v[j, pl.ds(g * L, L)]
                    cv = cidx_v[j, pl.ds(g * L, L)]
                    dvals = plsc.load_gather(dinv_v, [rv >> seven, rv & low])
                    plsc.addupdate_scatter(t_v, [cv >> seven, cv & low],
                                           dvals)
            return carry
        lax.fori_loop(0, chw, chunk, 0)
        plsc.subcore_barrier()
        pltpu.sync_copy(acc_s.at[my], acc_out.at[cid, my])
    pltpu.sync_copy(t_v, t_out.at[wid])


def _spmm2_body(n_pad, ch0, ch1,
                us_hbm, row_hbm, col_hbm, o_out,
                us_v, ridx_v, cidx_v, o0_v, o1_v):
    cid = lax.axis_index("c")
    sid = lax.axis_index("s")
    wid = sid * NC + cid
    chw = jnp.where(cid == 0, ch0, ch1)
    roff = _chunk_off(cid, sid, ch0, ch1)
    pltpu.sync_copy(us_hbm, us_v)
    pltpu.sync_copy(row_hbm.at[pl.ds(roff, ridx_v.shape[0])], ridx_v)
    pltpu.sync_copy(col_hbm.at[pl.ds(roff, cidx_v.shape[0])], cidx_v)
    _zero2d(o0_v, n_pad // 128)
    _zero2d(o1_v, n_pad // 128)

    two = jnp.full((L,), 2, jnp.int32)
    one = jnp.full((L,), 1, jnp.int32)
    seven = jnp.full((L,), 7, jnp.int32)
    low = jnp.full((L,), 127, jnp.int32)

    def chunk(j, carry):
        for g in range(CHUNK // L):
            rv = ridx_v[j, pl.ds(g * L, L)]
            cv = cidx_v[j, pl.ds(g * L, L)]
            i0 = rv * two
            v0 = plsc.load_gather(us_v, [i0])
            v1 = plsc.load_gather(us_v, [i0 + one])
            chi = cv >> seven
            clo = cv & low
            plsc.addupdate_scatter(o0_v, [chi, clo], v0)
            plsc.addupdate_scatter(o1_v, [chi, clo], v1)
        return carry
    lax.fori_loop(0, chw, chunk, 0)
    pltpu.sync_copy(o0_v, o_out.at[wid, 0])
    pltpu.sync_copy(o1_v, o_out.at[wid, 1])


def _dinv_body(degp_ref, dinv_ref):
    deg = jnp.sum(degp_ref[...], axis=0)
    dinv_ref[...] = jnp.where(deg > 0, lax.rsqrt(deg), 0.0)


def _xws_body(n_real, x_ref, dinvc_ref, w1a_ref, w1b_ref,
              xws_lo_ref, xws_hi_ref, c_ref, sum_ref):
    i = pl.program_id(0)
    xb = x_ref[...]
    xw = jnp.dot(xb, w1a_ref[...], preferred_element_type=jnp.float32)
    xws = dinvc_ref[...] * xw
    half = xws.shape[1] // 2
    xws_lo_ref[...] = xws[:, :half]
    xws_hi_ref[...] = xws[:, half:]
    s = jnp.sum(xb, axis=0, keepdims=True)

    @pl.when(i == 0)
    def _():
        sum_ref[...] = s

    @pl.when(i > 0)
    def _():
        sum_ref[...] = sum_ref[...] + s

    @pl.when(i == pl.num_programs(0) - 1)
    def _():
        c_ref[...] = jnp.dot(sum_ref[...] * (1.0 / n_real), w1b_ref[...],
                             preferred_element_type=jnp.float32)


def _dt_body(tp_ref, dinv_ref, dt_ref):
    dt_ref[...] = dinv_ref[...] * jnp.sum(tp_ref[...], axis=0)


def _h1_body(alo0_ref, alo1_ref, ahi0_ref, ahi1_ref, dinvc_ref, dtc_ref,
             c_ref, b1_ref, w2a_ref, w2b_ref, us_ref):
    d = dinvc_ref[...]
    dtc = dtc_ref[...]
    cv = c_ref[...]
    b1v = b1_ref[...]
    half = cv.shape[1] // 2
    h1_lo = jnp.maximum(
        d * (alo0_ref[...] + alo1_ref[...]) + dtc * cv[:, :half]
        + b1v[:, :half], 0.0)
    h1_hi = jnp.maximum(
        d * (ahi0_ref[...] + ahi1_ref[...]) + dtc * cv[:, half:]
        + b1v[:, half:], 0.0)
    u = (jnp.dot(h1_lo, w2a_ref[...], preferred_element_type=jnp.float32)
         + jnp.dot(h1_hi, w2b_ref[...], preferred_element_type=jnp.float32))
    us_ref[...] = d * u


def _final_body(op_ref, dinv_ref, b2b_ref, out_ref):
    s = jnp.sum(op_ref[...], axis=0)
    d = dinv_ref[...]
    out_ref[...] = d[None] * s + b2b_ref[...]


def kernel(x, edge_index, W1, b1, W2, b2):
    n, f = x.shape
    hid = W1.shape[1]
    e = edge_index.shape[1]

    n_pad = ((n + 1 + 255) // 256) * 256
    rb = 2048
    nrows = n_pad // 128

    ch_sum = -(-e // (NS * CHUNK))
    ch0 = int(round(ch_sum * CH0_FRAC))
    ch1 = ch_sum - ch0
    ch_max = max(ch0, ch1)
    tot_rows = NS * (ch0 + ch1) + ch_max
    npad_e = tot_rows * CHUNK - e

    def _layout(v):
        return jnp.concatenate(
            [v, jnp.full((npad_e,), n, jnp.int32)]).reshape(tot_rows, CHUNK)

    row_w = _layout(edge_index[0].astype(jnp.int32))
    col_w = _layout(edge_index[1].astype(jnp.int32))
    x_pad = jnp.concatenate([x, jnp.zeros((n_pad - n, f), x.dtype)])
    W1a, W1b = W1[:f], W1[f:]
    half = hid // 2
    W2a, W2b = W2[:half], W2[half:]
    zeros_blk = jnp.zeros((n_pad // NS, half), jnp.float32)

    mesh = _spmm_mesh()
    sc_params = pltpu.CompilerParams(needs_layout_passes=False,
                                     use_tc_tiling_on_sc=False)

    deg_part = pl.kernel(
        lambda *a: _deg_body(n_pad, ch0, ch1, *a),
        out_type=jax.ShapeDtypeStruct((NW, nrows, 128), jnp.float32),
        mesh=mesh,
        scratch_types=[pltpu.VMEM((ch_max, CHUNK), jnp.int32),
                       pltpu.VMEM((nrows, 128), jnp.float32)],
        compiler_params=sc_params,
    )(col_w)

    dinv = pl.pallas_call(
        _dinv_body,
        out_shape=jax.ShapeDtypeStruct((nrows, 128), jnp.float32),
    )(deg_part)
    dinv_col = dinv.reshape(n_pad, 1)

    xws_lo, xws_hi, c = pl.pallas_call(
        lambda *a: _xws_body(float(n), *a),
        grid=(n_pad // rb,),
        in_specs=[pl.BlockSpec((rb, f), lambda i: (i, 0)),
                  pl.BlockSpec((rb, 1), lambda i: (i, 0)),
                  pl.BlockSpec((f, hid), lambda i: (0, 0)),
                  pl.BlockSpec((f, hid), lambda i: (0, 0))],
        out_specs=[pl.BlockSpec((rb, half), lambda i: (i, 0)),
                   pl.BlockSpec((rb, half), lambda i: (i, 0)),
                   pl.BlockSpec((1, hid), lambda i: (0, 0))],
        out_shape=[jax.ShapeDtypeStruct((n_pad, half), jnp.float32),
                   jax.ShapeDtypeStruct((n_pad, half), jnp.float32),
                   jax.ShapeDtypeStruct((1, hid), jnp.float32)],
        scratch_shapes=[pltpu.VMEM((1, f), jnp.float32)],
    )(x_pad, dinv_col, W1a, W1b)

    acc_lo, acc_hi, t_part = pl.kernel(
        lambda *a: _spmm1_body(n_pad, ch0, ch1, *a),
        out_type=(jax.ShapeDtypeStruct((NC, n_pad, half), jnp.float32),
                  jax.ShapeDtypeStruct((NC, n_pad, half), jnp.float32),
                  jax.ShapeDtypeStruct((NW, nrows, 128), jnp.float32)),
        mesh=mesh,
        scratch_types=[pltpu.VMEM((ch_max, CHUNK), jnp.int32),
                       pltpu.VMEM((ch_max, CHUNK), jnp.int32),
                       pltpu.VMEM((nrows, 128), jnp.float32),
                       pltpu.VMEM((nrows, 128), jnp.float32),
                       pltpu.VMEM((CHUNK, half), jnp.float32),
                       pltpu.VMEM_SHARED((n_pad, half), jnp.float32),
                       pltpu.SemaphoreType.DMA],
        compiler_params=sc_params,
    )(xws_lo, xws_hi, row_w, col_w, dinv, zeros_blk)

    dt = pl.pallas_call(
        _dt_body,
        out_shape=jax.ShapeDtypeStruct((nrows, 128), jnp.float32),
    )(t_part, dinv)

    us = pl.pallas_call(
        _h1_body,
        grid=(n_pad // rb,),
        in_specs=[pl.BlockSpec((rb, half), lambda i: (i, 0)),
                  pl.BlockSpec((rb, half), lambda i: (i, 0)),
                  pl.BlockSpec((rb, half), lambda i: (i, 0)),
                  pl.BlockSpec((rb, half), lambda i: (i, 0)),
                  pl.BlockSpec((rb, 1), lambda i: (i, 0)),
                  pl.BlockSpec((rb, 1), lambda i: (i, 0)),
                  pl.BlockSpec((1, hid), lambda i: (0, 0)),
                  pl.BlockSpec((1, hid), lambda i: (0, 0)),
                  pl.BlockSpec((half, 2), lambda i: (0, 0)),
                  pl.BlockSpec((half, 2), lambda i: (0, 0))],
        out_specs=pl.BlockSpec((rb, 2), lambda i: (i, 0)),
        out_shape=jax.ShapeDtypeStruct((n_pad, 2), jnp.float32),
    )(acc_lo[0], acc_lo[1], acc_hi[0], acc_hi[1], dinv_col,
      dt.reshape(n_pad, 1), c, b1.reshape(1, hid), W2a, W2b)

    o_part = pl.kernel(
        lambda *a: _spmm2_body(n_pad, ch0, ch1, *a),
        out_type=jax.ShapeDtypeStruct((NW, 2, nrows, 128), jnp.float32),
        mesh=mesh,
        scratch_types=[pltpu.VMEM((2 * n_pad,), jnp.float32),
                       pltpu.VMEM((ch_max, CHUNK), jnp.int32),
                       pltpu.VMEM((ch_max, CHUNK), jnp.int32),
                       pltpu.VMEM((nrows, 128), jnp.float32),
                       pltpu.VMEM((nrows, 128), jnp.float32)],
        compiler_params=sc_params,
    )(us.reshape(2 * n_pad), row_w, col_w)

    b2b = jnp.broadcast_to(b2.reshape(2, 1, 1), (2, nrows, 128))
    out2 = pl.pallas_call(
        _final_body,
        out_shape=jax.ShapeDtypeStruct((2, nrows, 128), jnp.float32),
    )(o_part, dinv, b2b)

    return out2.reshape(2, n_pad)[:, :n].T
